# baseline (speedup 1.0000x reference)
import functools
import math

import jax
import jax.numpy as jnp
from jax import lax
from jax.experimental import pallas as pl
from jax.experimental.pallas import tpu as pltpu

N_NODES = 10000
N_EDGES = 320000
D_FEAT = 128
HID = 64
N_GRAPHS = 16

NEG_BIG = -3.0e38


_SQRT_HALF = 0.7071067811865476


def _gelu(x):
    return 0.5 * x * (1.0 + lax.erf(x * _SQRT_HALF))


def _ln(x, g, b, eps=1e-5):
    mu = jnp.mean(x, axis=-1, keepdims=True)
    var = jnp.mean((x - mu) ** 2, axis=-1, keepdims=True)
    return (x - mu) * jax.lax.rsqrt(var + eps) * g + b


def _edge_mlp_body(src, dest, ea, w1s, w1d, w1e, b1, w2, b2, g, be, out,
                   *, residual):
    z = jnp.dot(src[...], w1s[...], preferred_element_type=jnp.float32, precision=lax.Precision.HIGHEST)
    z += jnp.dot(dest[...], w1d[...], preferred_element_type=jnp.float32, precision=lax.Precision.HIGHEST)
    z += jnp.dot(ea[...], w1e[...], preferred_element_type=jnp.float32, precision=lax.Precision.HIGHEST)
    z += b1[...]
    z = _gelu(z)
    z = jnp.dot(z, w2[...], preferred_element_type=jnp.float32, precision=lax.Precision.HIGHEST) + b2[...]
    z = _ln(z, g[...], be[...])
    if residual:
        z = z + ea[...]
    out[...] = z


def edge_mlp(src, dest, ea, w1s, w1d, w1e, b1, w2, b2, g, be, *, residual,
             block_e=2000):
    E = src.shape[0]
    grid = (E // block_e,)
    din = src.shape[1]
    ein = ea.shape[1]

    def rowblk(width):
        return pl.BlockSpec((block_e, width), lambda i: (i, 0))

    def whole(a):
        return pl.BlockSpec(a.shape, lambda i: tuple(0 for _ in a.shape))

    return pl.pallas_call(
        functools.partial(_edge_mlp_body, residual=residual),
        grid=grid,
        in_specs=[
            rowblk(din), rowblk(din), rowblk(ein),
            whole(w1s), whole(w1d), whole(w1e), whole(b1),
            whole(w2), whole(b2), whole(g), whole(be),
        ],
        out_specs=rowblk(HID),
        out_shape=jax.ShapeDtypeStruct((E, HID), jnp.float32),
    )(src, dest, ea, w1s, w1d, w1e, b1, w2, b2, g, be)


def _node_mlp_body(h, s, mx, cnt, batch, urow, w1h, w1s, w1m, w1mn, w1u, b1,
                   w2, b2, g, be, out, *, residual):
    cntc = cnt[...][:, 0:1]
    has = cntc > 0.0
    sv = s[...]
    mxv = jnp.where(has, mx[...], 0.0)
    mean = sv / jnp.maximum(cntc, 1.0)
    bq = batch[...]
    G = urow.shape[1]
    gi = lax.broadcasted_iota(jnp.int32, (bq.shape[0], G), 1)
    ub = jnp.sum(jnp.where(bq == gi, urow[...], 0.0), axis=1, keepdims=True)
    z = jnp.dot(h[...], w1h[...], preferred_element_type=jnp.float32, precision=lax.Precision.HIGHEST)
    z += jnp.dot(sv, w1s[...], preferred_element_type=jnp.float32, precision=lax.Precision.HIGHEST)
    z += jnp.dot(mxv, w1m[...], preferred_element_type=jnp.float32, precision=lax.Precision.HIGHEST)
    z += jnp.dot(mean, w1mn[...], preferred_element_type=jnp.float32, precision=lax.Precision.HIGHEST)
    z += ub * w1u[...]
    z += b1[...]
    z = _gelu(z)
    z = jnp.dot(z, w2[...], preferred_element_type=jnp.float32, precision=lax.Precision.HIGHEST) + b2[...]
    z = _ln(z, g[...], be[...])
    if residual:
        z = z + h[...]
    out[...] = z


def node_mlp(h, s, mx, cnt, batch2d, urow, w1h, w1s, w1m, w1mn, w1u, b1, w2,
             b2, g, be, *, residual, block_n=1000):
    N = h.shape[0]
    grid = (N // block_n,)
    din = h.shape[1]

    def rowblk(width, dtype=jnp.float32):
        return pl.BlockSpec((block_n, width), lambda i: (i, 0))

    def whole(a):
        return pl.BlockSpec(a.shape, lambda i: tuple(0 for _ in a.shape))

    return pl.pallas_call(
        functools.partial(_node_mlp_body, residual=residual),
        grid=grid,
        in_specs=[
            rowblk(din), rowblk(HID), rowblk(HID), rowblk(cnt.shape[1]),
            rowblk(1), whole(urow),
            whole(w1h), whole(w1s), whole(w1m), whole(w1mn), whole(w1u),
            whole(b1), whole(w2), whole(b2), whole(g), whole(be),
        ],
        out_specs=rowblk(HID),
        out_shape=jax.ShapeDtypeStruct((N, HID), jnp.float32),
    )(h, s, mx, cnt, batch2d, urow, w1h, w1s, w1m, w1mn, w1u, b1, w2, b2, g,
      be)


def _pool_out_body(h, batch, u, w1a, w1b, w1c, w1u, b1, w2, b2, w3, b3, w4,
                   b4, out, sum_acc, max_acc, cnt_acc, *, nblocks):
    i = pl.program_id(0)

    @pl.when(i == 0)
    def _init():
        sum_acc[...] = jnp.zeros_like(sum_acc)
        max_acc[...] = jnp.full_like(max_acc, NEG_BIG)
        cnt_acc[...] = jnp.zeros_like(cnt_acc)

    hv = h[...]
    bq = batch[...]
    B = hv.shape[0]
    G = sum_acc.shape[0]
    onehot = (bq == lax.broadcasted_iota(jnp.int32, (B, G), 1)).astype(
        jnp.float32)
    sum_acc[...] += lax.dot_general(
        onehot, hv, (((0,), (0,)), ((), ())),
        preferred_element_type=jnp.float32, precision=lax.Precision.HIGHEST)
    cnt_acc[...] += jnp.sum(onehot, axis=0, keepdims=True)
    mcur = max_acc[...]
    newmax = []
    for gidx in range(G):
        m = jnp.max(jnp.where(bq == gidx, hv, NEG_BIG), axis=0)
        newmax.append(jnp.maximum(mcur[gidx], m))
    max_acc[...] = jnp.stack(newmax, axis=0)

    @pl.when(i == nblocks - 1)
    def _finish():
        addp = sum_acc[...]
        cg = cnt_acc[...][0, :][:, None]
        meanp = addp / jnp.maximum(cg, 1.0)
        maxp = jnp.where(cg > 0.0, max_acc[...], 0.0)
        uv = u[...]
        z = jnp.dot(addp, w1a[...], preferred_element_type=jnp.float32, precision=lax.Precision.HIGHEST)
        z += jnp.dot(meanp, w1b[...], preferred_element_type=jnp.float32, precision=lax.Precision.HIGHEST)
        z += jnp.dot(maxp, w1c[...], preferred_element_type=jnp.float32, precision=lax.Precision.HIGHEST)
        z += uv * w1u[...]
        z += b1[...]
        z = _gelu(z)
        z = jnp.dot(z, w2[...], preferred_element_type=jnp.float32, precision=lax.Precision.HIGHEST) + b2[...]
        z = _gelu(z)
        z = jnp.dot(z, w3[...], preferred_element_type=jnp.float32, precision=lax.Precision.HIGHEST) + b3[...]
        z = _gelu(z)
        z = jnp.dot(z, w4[...], preferred_element_type=jnp.float32, precision=lax.Precision.HIGHEST) + b4[...]
        z = jax.nn.softplus(z)
        col = lax.broadcasted_iota(jnp.int32, z.shape, 1)
        z = jnp.where(col == 1, 0.85 * z, z)
        out[...] = z


def pool_out(h, batch2d, u, op, *, block_n=1000):
    N = h.shape[0]
    nblocks = N // block_n
    w1 = op['W1']
    w1a = w1[0:HID]
    w1b = w1[HID:2 * HID]
    w1c = w1[2 * HID:3 * HID]
    w1u = w1[3 * HID:3 * HID + 1]

    def rowblk(width):
        return pl.BlockSpec((block_n, width), lambda i: (i, 0))

    def whole(a):
        return pl.BlockSpec(a.shape, lambda i: tuple(0 for _ in a.shape))

    return pl.pallas_call(
        functools.partial(_pool_out_body, nblocks=nblocks),
        grid=(nblocks,),
        in_specs=[
            rowblk(HID), rowblk(1), whole(u),
            whole(w1a), whole(w1b), whole(w1c), whole(w1u), whole(op['b1']),
            whole(op['W2']), whole(op['b2']), whole(op['W3']), whole(op['b3']),
            whole(op['W4']), whole(op['b4']),
        ],
        out_specs=pl.BlockSpec((N_GRAPHS, 2), lambda i: (0, 0)),
        out_shape=jax.ShapeDtypeStruct((N_GRAPHS, 2), jnp.float32),
        scratch_shapes=[
            pltpu.VMEM((N_GRAPHS, HID), jnp.float32),
            pltpu.VMEM((N_GRAPHS, HID), jnp.float32),
            pltpu.VMEM((1, N_GRAPHS), jnp.float32),
        ],
    )(h, batch2d, u, w1a, w1b, w1c, w1u, op['b1'], op['W2'], op['b2'],
      op['W3'], op['b3'], op['W4'], op['b4'])


def kernel(x, edge_attr, u, params, edge_index, batch):
    row = edge_index[0]
    col = edge_index[1]
    batch2d = batch.astype(jnp.int32).reshape(N_NODES, 1)
    urow = u.reshape(1, N_GRAPHS)

    cnt = jax.ops.segment_sum(
        jnp.ones((N_EDGES, 1), jnp.float32), col, num_segments=N_NODES)
    cnt8 = jnp.broadcast_to(cnt, (N_NODES, 8))

    h = x
    ea = edge_attr
    for l, lp in enumerate(params['layers']):
        res = l > 0
        din = h.shape[1]
        src = jnp.take(h, row, axis=0)
        dest = jnp.take(h, col, axis=0)
        ep = lp['edge']
        w1 = ep['W1']
        w1s = w1[0:din]
        w1d = w1[din:2 * din]
        w1e = w1[2 * din:]
        ein = w1e.shape[0]
        if ein < 8:
            w1e = jnp.pad(w1e, ((0, 8 - ein), (0, 0)))
            eain = jnp.pad(ea, ((0, 0), (0, 8 - ein)))
        else:
            eain = ea
        e = edge_mlp(src, dest, eain, w1s, w1d, w1e,
                     ep['b1'].reshape(1, HID), ep['W2'],
                     ep['b2'].reshape(1, HID), ep['g'].reshape(1, HID),
                     ep['be'].reshape(1, HID), residual=res)
        ea = e

        s = jax.ops.segment_sum(ea, col, num_segments=N_NODES)
        mx = jax.ops.segment_max(ea, col, num_segments=N_NODES)
        mx = jnp.where(cnt > 0, mx, NEG_BIG)

        np_ = lp['node']
        w1n = np_['W1']
        w1h = w1n[0:din]
        w1sa = w1n[din:din + HID]
        w1m = w1n[din + HID:din + 2 * HID]
        w1mn = w1n[din + 2 * HID:din + 3 * HID]
        w1u = w1n[din + 3 * HID:din + 3 * HID + 1]
        h = node_mlp(h, s, mx, cnt8, batch2d, urow, w1h, w1sa, w1m, w1mn,
                     w1u, np_['b1'].reshape(1, HID), np_['W2'],
                     np_['b2'].reshape(1, HID), np_['g'].reshape(1, HID),
                     np_['be'].reshape(1, HID), residual=res)

    return pool_out(h, batch2d, u, params['out'])


# speedup vs baseline: 1.6591x; 1.6591x over previous
import functools
import math

import jax
import jax.numpy as jnp
from jax import lax
from jax.experimental import pallas as pl
from jax.experimental.pallas import tpu as pltpu
from jax.experimental.pallas import tpu_sc as plsc

N_NODES = 10000
N_EDGES = 320000
D_FEAT = 128
HID = 64
N_GRAPHS = 16

NEG_BIG = -3.0e38
_SQRT_HALF = 0.7071067811865476

_HIGH = lax.Precision.HIGHEST


def _gelu(x):
    return 0.5 * x * (1.0 + lax.erf(x * _SQRT_HALF))


def _ln(x, g, b, eps=1e-5):
    mu = jnp.mean(x, axis=-1, keepdims=True)
    var = jnp.mean((x - mu) ** 2, axis=-1, keepdims=True)
    return (x - mu) * jax.lax.rsqrt(var + eps) * g + b


def _dot(a, b):
    return jnp.dot(a, b, preferred_element_type=jnp.float32, precision=_HIGH)


_SC_CHUNK = 200
_SC_GRP = 25
_SC_NC = 2
_SC_NS = 16
_TW = 2 * HID


def sc_gather_add(t1, t2, row, col):
    E = N_EDGES
    nw = _SC_NC * _SC_NS
    per_w = E // nw
    nchunks = per_w // _SC_CHUNK
    ngrp = _SC_CHUNK // _SC_GRP
    grp_per_w = per_w // _SC_GRP
    mesh = plsc.VectorSubcoreMesh(core_axis_name="c", subcore_axis_name="s")

    row2 = row.reshape(E // _SC_GRP, _SC_GRP)
    col2 = col.reshape(E // _SC_GRP, _SC_GRP)

    @functools.partial(
        pl.kernel, mesh=mesh,
        out_type=jax.ShapeDtypeStruct((E, _TW), jnp.float32),
        scratch_types=[
            pltpu.VMEM((ngrp, _SC_GRP), jnp.int32),
            pltpu.VMEM((_SC_CHUNK, _TW), jnp.float32),
            pltpu.SemaphoreType.DMA,
        ],
    )
    def k(t1_hbm, t2_hbm, row_hbm, col_hbm, out_hbm, idx_v, rows_v, sem):
        wid = lax.axis_index("s") * _SC_NC + lax.axis_index("c")
        base = wid * per_w
        gbase = wid * grp_per_w

        def chunk_body(ci, _):
            off = base + ci * _SC_CHUNK
            goff = gbase + ci * ngrp
            pltpu.sync_copy(row_hbm.at[pl.ds(goff, ngrp)], idx_v)
            copies = []
            for g in range(ngrp):
                copies.append(pltpu.async_copy(
                    t1_hbm.at[idx_v.at[g]],
                    rows_v.at[pl.ds(g * _SC_GRP, _SC_GRP)], sem))
            for c in copies:
                c.wait()
            pltpu.sync_copy(col_hbm.at[pl.ds(goff, ngrp)], idx_v)
            copies = []
            for g in range(ngrp):
                copies.append(pltpu.async_copy(
                    t2_hbm.at[idx_v.at[g]],
                    rows_v.at[pl.ds(g * _SC_GRP, _SC_GRP)], sem, add=True))
            for c in copies:
                c.wait()
            pltpu.sync_copy(rows_v, out_hbm.at[pl.ds(off, _SC_CHUNK)])
            return _

        lax.fori_loop(0, nchunks, chunk_body, 0)

    return k(t1, t2, row2, col2)


def _edge_mlp_body(gsum, ea, w1e, b1, w2, b2, g, be, out, *, residual):
    z = gsum[...][:, 0:HID] + _dot(ea[...], w1e[...]) + b1[...]
    z = _gelu(z)
    z = _dot(z, w2[...]) + b2[...]
    z = _ln(z, g[...], be[...])
    if residual:
        z = z + ea[...]
    out[...] = z


def edge_mlp(gsum, ea, w1e, b1, w2, b2, g, be, *, residual, block_e=2000):
    E = gsum.shape[0]
    grid = (E // block_e,)
    ein = ea.shape[1]

    def rowblk(width):
        return pl.BlockSpec((block_e, width), lambda i: (i, 0))

    def whole(a):
        return pl.BlockSpec(a.shape, lambda i: tuple(0 for _ in a.shape))

    return pl.pallas_call(
        functools.partial(_edge_mlp_body, residual=residual),
        grid=grid,
        in_specs=[
            rowblk(_TW), rowblk(ein),
            whole(w1e), whole(b1), whole(w2), whole(b2), whole(g), whole(be),
        ],
        out_specs=rowblk(HID),
        out_shape=jax.ShapeDtypeStruct((E, HID), jnp.float32),
    )(gsum, ea, w1e, b1, w2, b2, g, be)


def _node_mlp_body(h, s, mx, cnt, batch, urow, w1h, w1s, w1m, w1mn, w1u, b1,
                   w2, b2, g, be, wns, wnd, out, hs_out, hd_out, *,
                   residual, project):
    cntc = cnt[...][:, 0:1]
    has = cntc > 0.0
    sv = s[...]
    mxv = jnp.where(has, mx[...], 0.0)
    mean = sv / jnp.maximum(cntc, 1.0)
    bq = batch[...]
    G = urow.shape[1]
    gi = lax.broadcasted_iota(jnp.int32, (bq.shape[0], G), 1)
    ub = jnp.sum(jnp.where(bq == gi, urow[...], 0.0), axis=1, keepdims=True)
    z = _dot(h[...], w1h[...])
    z += _dot(sv, w1s[...])
    z += _dot(mxv, w1m[...])
    z += _dot(mean, w1mn[...])
    z += ub * w1u[...]
    z += b1[...]
    z = _gelu(z)
    z = _dot(z, w2[...]) + b2[...]
    z = _ln(z, g[...], be[...])
    if residual:
        z = z + h[...]
    out[...] = z
    if project:
        zs = _dot(z, wns[...])
        zd = _dot(z, wnd[...])
        hs_out[...] = jnp.concatenate([zs, zd], axis=-1)
        hd_out[...] = jnp.concatenate([zd, zs], axis=-1)


def node_mlp(h, s, mx, cnt, batch2d, urow, w1h, w1s, w1m, w1mn, w1u, b1, w2,
             b2, g, be, wns, wnd, *, residual, project, block_n=1000):
    N = h.shape[0]
    grid = (N // block_n,)
    din = h.shape[1]

    def rowblk(width):
        return pl.BlockSpec((block_n, width), lambda i: (i, 0))

    def whole(a):
        return pl.BlockSpec(a.shape, lambda i: tuple(0 for _ in a.shape))

    out_specs = [rowblk(HID), rowblk(_TW), rowblk(_TW)]
    out_shape = [jax.ShapeDtypeStruct((N, HID), jnp.float32),
                 jax.ShapeDtypeStruct((N, _TW), jnp.float32),
                 jax.ShapeDtypeStruct((N, _TW), jnp.float32)]
    return pl.pallas_call(
        functools.partial(_node_mlp_body, residual=residual, project=project),
        grid=grid,
        in_specs=[
            rowblk(din), rowblk(HID), rowblk(HID), rowblk(cnt.shape[1]),
            rowblk(1), whole(urow),
            whole(w1h), whole(w1s), whole(w1m), whole(w1mn), whole(w1u),
            whole(b1), whole(w2), whole(b2), whole(g), whole(be),
            whole(wns), whole(wnd),
        ],
        out_specs=out_specs,
        out_shape=out_shape,
    )(h, s, mx, cnt, batch2d, urow, w1h, w1s, w1m, w1mn, w1u, b1, w2, b2, g,
      be, wns, wnd)


def _proj_body(h, ws, wd, t1_out, t2_out):
    hv = h[...]
    zs = _dot(hv, ws[...])
    zd = _dot(hv, wd[...])
    t1_out[...] = jnp.concatenate([zs, zd], axis=-1)
    t2_out[...] = jnp.concatenate([zd, zs], axis=-1)


def proj(h, ws, wd, *, block_n=1000):
    N = h.shape[0]
    din = h.shape[1]

    def rowblk(width):
        return pl.BlockSpec((block_n, width), lambda i: (i, 0))

    def whole(a):
        return pl.BlockSpec(a.shape, lambda i: tuple(0 for _ in a.shape))

    return pl.pallas_call(
        _proj_body,
        grid=(N // block_n,),
        in_specs=[rowblk(din), whole(ws), whole(wd)],
        out_specs=[rowblk(_TW), rowblk(_TW)],
        out_shape=[jax.ShapeDtypeStruct((N, _TW), jnp.float32)] * 2,
    )(h, ws, wd)


def _pool_out_body(h, batch, u, w1a, w1b, w1c, w1u, b1, w2, b2, w3, b3, w4,
                   b4, out, sum_acc, max_acc, cnt_acc, *, nblocks):
    i = pl.program_id(0)

    @pl.when(i == 0)
    def _init():
        sum_acc[...] = jnp.zeros_like(sum_acc)
        max_acc[...] = jnp.full_like(max_acc, NEG_BIG)
        cnt_acc[...] = jnp.zeros_like(cnt_acc)

    hv = h[...]
    bq = batch[...]
    B = hv.shape[0]
    G = sum_acc.shape[0]
    onehot = (bq == lax.broadcasted_iota(jnp.int32, (B, G), 1)).astype(
        jnp.float32)
    sum_acc[...] += lax.dot_general(
        onehot, hv, (((0,), (0,)), ((), ())),
        preferred_element_type=jnp.float32, precision=_HIGH)
    cnt_acc[...] += jnp.sum(onehot, axis=0, keepdims=True)
    mcur = max_acc[...]
    newmax = []
    for gidx in range(G):
        m = jnp.max(jnp.where(bq == gidx, hv, NEG_BIG), axis=0)
        newmax.append(jnp.maximum(mcur[gidx], m))
    max_acc[...] = jnp.stack(newmax, axis=0)

    @pl.when(i == nblocks - 1)
    def _finish():
        addp = sum_acc[...]
        cg = cnt_acc[...][0, :][:, None]
        meanp = addp / jnp.maximum(cg, 1.0)
        maxp = jnp.where(cg > 0.0, max_acc[...], 0.0)
        uv = u[...]
        z = _dot(addp, w1a[...]) + _dot(meanp, w1b[...]) + _dot(maxp, w1c[...])
        z += uv * w1u[...]
        z += b1[...]
        z = _gelu(z)
        z = _dot(z, w2[...]) + b2[...]
        z = _gelu(z)
        z = _dot(z, w3[...]) + b3[...]
        z = _gelu(z)
        z = _dot(z, w4[...]) + b4[...]
        z = jax.nn.softplus(z)
        col = lax.broadcasted_iota(jnp.int32, z.shape, 1)
        z = jnp.where(col == 1, 0.85 * z, z)
        out[...] = z


def pool_out(h, batch2d, u, op, *, block_n=1000):
    N = h.shape[0]
    nblocks = N // block_n
    w1 = op['W1']
    w1a = w1[0:HID]
    w1b = w1[HID:2 * HID]
    w1c = w1[2 * HID:3 * HID]
    w1u = w1[3 * HID:3 * HID + 1]

    def rowblk(width):
        return pl.BlockSpec((block_n, width), lambda i: (i, 0))

    def whole(a):
        return pl.BlockSpec(a.shape, lambda i: tuple(0 for _ in a.shape))

    return pl.pallas_call(
        functools.partial(_pool_out_body, nblocks=nblocks),
        grid=(nblocks,),
        in_specs=[
            rowblk(HID), rowblk(1), whole(u),
            whole(w1a), whole(w1b), whole(w1c), whole(w1u), whole(op['b1']),
            whole(op['W2']), whole(op['b2']), whole(op['W3']), whole(op['b3']),
            whole(op['W4']), whole(op['b4']),
        ],
        out_specs=pl.BlockSpec((N_GRAPHS, 2), lambda i: (0, 0)),
        out_shape=jax.ShapeDtypeStruct((N_GRAPHS, 2), jnp.float32),
        scratch_shapes=[
            pltpu.VMEM((N_GRAPHS, HID), jnp.float32),
            pltpu.VMEM((N_GRAPHS, HID), jnp.float32),
            pltpu.VMEM((1, N_GRAPHS), jnp.float32),
        ],
    )(h, batch2d, u, w1a, w1b, w1c, w1u, op['b1'], op['W2'], op['b2'],
      op['W3'], op['b3'], op['W4'], op['b4'])


def kernel(x, edge_attr, u, params, edge_index, batch):
    edge_index = edge_index.astype(jnp.int32)
    row = edge_index[0]
    col = edge_index[1]
    batch2d = batch.astype(jnp.int32).reshape(N_NODES, 1)
    urow = u.reshape(1, N_GRAPHS)

    cnt = jax.ops.segment_sum(
        jnp.ones((N_EDGES, 1), jnp.float32), col, num_segments=N_NODES)
    cnt8 = jnp.broadcast_to(cnt, (N_NODES, 8))

    h = x
    ea = edge_attr
    t1 = t2 = None
    for l, lp in enumerate(params['layers']):
        res = l > 0
        din = h.shape[1]
        ep = lp['edge']
        w1 = ep['W1']
        if l == 0:
            t1, t2 = proj(h, w1[0:din], w1[din:2 * din])
        w1e = w1[2 * din:]
        ein = w1e.shape[0]
        if ein < 8:
            w1e = jnp.pad(w1e, ((0, 8 - ein), (0, 0)))
            eain = jnp.pad(ea, ((0, 0), (0, 8 - ein)))
        else:
            eain = ea

        gsum = sc_gather_add(t1, t2, row, col)

        e = edge_mlp(gsum, eain, w1e,
                     ep['b1'].reshape(1, HID), ep['W2'],
                     ep['b2'].reshape(1, HID), ep['g'].reshape(1, HID),
                     ep['be'].reshape(1, HID), residual=res)
        ea = e

        s = jax.ops.segment_sum(ea, col, num_segments=N_NODES)
        mx = jax.ops.segment_max(ea, col, num_segments=N_NODES)
        mx = jnp.where(cnt > 0, mx, NEG_BIG)

        np_ = lp['node']
        w1n = np_['W1']
        project = l + 1 < len(params['layers'])
        if project:
            wn = params['layers'][l + 1]['edge']['W1']
            wns = wn[0:HID]
            wnd = wn[HID:2 * HID]
        else:
            wns = wnd = jnp.zeros((HID, HID), jnp.float32)
        h, t1, t2 = node_mlp(
            h, s, mx, cnt8, batch2d, urow,
            w1n[0:din], w1n[din:din + HID], w1n[din + HID:din + 2 * HID],
            w1n[din + 2 * HID:din + 3 * HID],
            w1n[din + 3 * HID:din + 3 * HID + 1],
            np_['b1'].reshape(1, HID), np_['W2'], np_['b2'].reshape(1, HID),
            np_['g'].reshape(1, HID), np_['be'].reshape(1, HID),
            wns, wnd, residual=res, project=project)

    return pool_out(h, batch2d, u, params['out'])
